# baseline (speedup 1.0000x reference)
import functools

import jax
import jax.numpy as jnp
from jax import lax
from jax.experimental import pallas as pl
from jax.experimental.pallas import tpu as pltpu
from jax.experimental.pallas import tpu_sc as plsc

MASK_FRACTION = 0.75
_V = 16


def _sc_mask_and_gather(noise_flat, x2, N, L, keep):
    D = x2.shape[1]
    info = plsc.get_sparse_core_info()
    nw = info.num_cores * info.num_subcores
    rows_w = N // nw
    nv = L // _V
    kv = keep // _V
    ch = 48
    nch = keep // ch
    mesh = plsc.VectorSubcoreMesh(core_axis_name="c", subcore_axis_name="s")

    @functools.partial(
        pl.kernel,
        mesh=mesh,
        compiler_params=pltpu.CompilerParams(needs_layout_passes=False),
        out_type=(
            jax.ShapeDtypeStruct((N * keep, D), jnp.float32),
            jax.ShapeDtypeStruct((N * L,), jnp.float32),
            jax.ShapeDtypeStruct((N * L,), jnp.int32),
        ),
        scratch_types=[
            pltpu.VMEM((rows_w * L,), jnp.float32),
            pltpu.VMEM((L,), jnp.int32),
            pltpu.VMEM((L,), jnp.int32),
            pltpu.VMEM((256,), jnp.int32),
            pltpu.VMEM((256,), jnp.int32),
            pltpu.VMEM((L,), jnp.int32),
            pltpu.VMEM((L,), jnp.int32),
            pltpu.VMEM((rows_w * keep,), jnp.int32),
            pltpu.VMEM((rows_w * L,), jnp.float32),
            pltpu.VMEM((rows_w * L,), jnp.int32),
            pltpu.VMEM((ch, D), jnp.float32),
            pltpu.VMEM((ch, D), jnp.float32),
            pltpu.VMEM((ch, D), jnp.float32),
            pltpu.SemaphoreType.DMA,
            pltpu.SemaphoreType.DMA,
            pltpu.SemaphoreType.DMA,
            pltpu.SemaphoreType.DMA,
            pltpu.SemaphoreType.DMA,
            pltpu.SemaphoreType.DMA,
        ],
    )
    def k(noise_hbm, x_hbm, xm_hbm, mask_hbm, rest_hbm,
          noise_v, ka, kb, hist_v, off_v, dig_v, cnt_v, ids_v, mask4_v, rest4_v,
          buf0, buf1, buf2, g0, g1, g2, o0, o1, o2):
        wid = lax.axis_index("s") * info.num_cores + lax.axis_index("c")
        row0 = wid * rows_w
        pltpu.sync_copy(noise_hbm.at[pl.ds(row0 * L, rows_w * L)], noise_v)

        bufs = (buf0, buf1, buf2)
        gsems = (g0, g1, g2)
        osems = (o0, o1, o2)

        def gather_cp(r, j):
            return pltpu.make_async_copy(
                x_hbm.at[ids_v.at[pl.ds(r * keep + j * ch, ch)]],
                bufs[j], gsems[j])

        def out_cp(r, j):
            return pltpu.make_async_copy(
                bufs[j],
                xm_hbm.at[pl.ds((row0 + r) * keep + j * ch, ch)],
                osems[j])

        def row_body(r, carry):
            ones = jnp.ones((_V,), jnp.int32)

            def radix_pass(src_dig, src_pay, dst):
                for h in range(256 // _V):
                    hist_v[pl.ds(h * _V, _V)] = jnp.zeros((_V,), jnp.int32)
                for c in range(nv):
                    plsc.addupdate_scatter(hist_v, [src_dig(c)], ones)
                carry_s = jnp.int32(0)
                for h in range(256 // _V):
                    hv = hist_v[pl.ds(h * _V, _V)]
                    inc = plsc.cumsum(hv)
                    off_v[pl.ds(h * _V, _V)] = inc - hv + carry_s
                    carry_s = carry_s + jnp.sum(hv)
                for c in range(nv):
                    d = src_dig(c)
                    cnt, _ = plsc.scan_count(d)
                    dig_v[pl.ds(c * _V, _V)] = d
                    cnt_v[pl.ds(c * _V, _V)] = cnt
                for c in range(nv):
                    d = dig_v[pl.ds(c * _V, _V)]
                    cnt = cnt_v[pl.ds(c * _V, _V)]
                    base = plsc.load_gather(off_v, [d])
                    plsc.store_scatter(dst, [base + cnt - 1], src_pay(c))
                    plsc.addupdate_scatter(off_v, [d], ones)

            def ki_of(c):
                nvv = noise_v[pl.ds(r * L + c * _V, _V)]
                return (nvv * 8388608.0).astype(jnp.int32)

            radix_pass(
                lambda c: ki_of(c) & 255,
                lambda c: ((ki_of(c) >> 8) << 10)
                | (lax.iota(jnp.int32, _V) + c * _V),
                ka)
            radix_pass(
                lambda c: (ka[pl.ds(c * _V, _V)] >> 10) & 255,
                lambda c: ka[pl.ds(c * _V, _V)],
                kb)

            @pl.when(r >= 1)
            def _():
                for j in range(nch):
                    gather_cp(r - 1, j).wait()
                    out_cp(r - 1, j).start()

            radix_pass(
                lambda c: kb[pl.ds(c * _V, _V)] >> 18,
                lambda c: kb[pl.ds(c * _V, _V)],
                ka)

            for q in range(nv):
                sv = ka[pl.ds(q * _V, _V)] & 1023
                kidx = lax.iota(jnp.int32, _V) + q * _V
                plsc.store_scatter(rest4_v, [sv + r * L], kidx)
                plsc.store_scatter(
                    mask4_v, [sv + r * L],
                    jnp.where(kidx >= keep, 1.0, 0.0).astype(jnp.float32))
            for q in range(kv):
                sv = ka[pl.ds(q * _V, _V)] & 1023
                ids_v[pl.ds(r * keep + q * _V, _V)] = sv + (row0 + r) * L

            @pl.when(r >= 1)
            def _():
                for j in range(nch):
                    out_cp(r - 1, j).wait()
            for j in range(nch):
                gather_cp(r, j).start()
            return carry

        lax.fori_loop(0, rows_w, row_body, 0)

        pltpu.sync_copy(mask4_v, mask_hbm.at[pl.ds(row0 * L, rows_w * L)])
        pltpu.sync_copy(rest4_v, rest_hbm.at[pl.ds(row0 * L, rows_w * L)])

        last = rows_w - 1
        for j in range(nch):
            gather_cp(last, j).wait()
            out_cp(last, j).start()
        for j in range(nch):
            out_cp(last, j).wait()

    return k(noise_flat, x2)


def kernel(x):
    N, L, D = x.shape
    keep = int(L * (1 - MASK_FRACTION))
    noise = jax.random.uniform(jax.random.key(42), (N, L), dtype=jnp.float32)
    xm, mask, rest = _sc_mask_and_gather(
        noise.reshape(N * L), x.reshape(N * L, D), N, L, keep)
    return (xm.reshape(N, keep, D),
            mask.reshape(N, L),
            rest.reshape(N, L))


# speedup vs baseline: 1.0045x; 1.0045x over previous
import functools

import jax
import jax.numpy as jnp
from jax import lax
from jax.experimental import pallas as pl
from jax.experimental.pallas import tpu as pltpu
from jax.experimental.pallas import tpu_sc as plsc

MASK_FRACTION = 0.75
_V = 16


def _sc_mask_and_gather(noise_flat, x2, N, L, keep):
    D = x2.shape[1]
    info = plsc.get_sparse_core_info()
    nw = info.num_cores * info.num_subcores
    rows_w = N // nw
    nv = L // _V
    kv = keep // _V
    ch = 48
    nch = keep // ch
    mesh = plsc.VectorSubcoreMesh(core_axis_name="c", subcore_axis_name="s")

    @functools.partial(
        pl.kernel,
        mesh=mesh,
        compiler_params=pltpu.CompilerParams(needs_layout_passes=False),
        out_type=(
            jax.ShapeDtypeStruct((N * keep, D), jnp.float32),
            jax.ShapeDtypeStruct((N * L,), jnp.float32),
            jax.ShapeDtypeStruct((N * L,), jnp.int32),
        ),
        scratch_types=[
            pltpu.VMEM((rows_w * L,), jnp.float32),
            pltpu.VMEM((L,), jnp.int32),
            pltpu.VMEM((L,), jnp.int32),
            pltpu.VMEM((256,), jnp.int32),
            pltpu.VMEM((256,), jnp.int32),
            pltpu.VMEM((rows_w * keep,), jnp.int32),
            pltpu.VMEM((rows_w * L,), jnp.float32),
            pltpu.VMEM((rows_w * L,), jnp.int32),
            pltpu.VMEM((ch, D), jnp.float32),
            pltpu.VMEM((ch, D), jnp.float32),
            pltpu.VMEM((ch, D), jnp.float32),
            pltpu.SemaphoreType.DMA,
            pltpu.SemaphoreType.DMA,
            pltpu.SemaphoreType.DMA,
            pltpu.SemaphoreType.DMA,
            pltpu.SemaphoreType.DMA,
            pltpu.SemaphoreType.DMA,
        ],
    )
    def k(noise_hbm, x_hbm, xm_hbm, mask_hbm, rest_hbm,
          noise_v, ka, kb, hist_v, off_v, ids_v, mask4_v, rest4_v,
          buf0, buf1, buf2, g0, g1, g2, o0, o1, o2):
        wid = lax.axis_index("s") * info.num_cores + lax.axis_index("c")
        row0 = wid * rows_w
        pltpu.sync_copy(noise_hbm.at[pl.ds(row0 * L, rows_w * L)], noise_v)

        bufs = (buf0, buf1, buf2)
        gsems = (g0, g1, g2)
        osems = (o0, o1, o2)

        def gather_cp(r, j):
            return pltpu.make_async_copy(
                x_hbm.at[ids_v.at[pl.ds(r * keep + j * ch, ch)]],
                bufs[j], gsems[j])

        def out_cp(r, j):
            return pltpu.make_async_copy(
                bufs[j],
                xm_hbm.at[pl.ds((row0 + r) * keep + j * ch, ch)],
                osems[j])

        def row_body(r, carry):
            ones = jnp.ones((_V,), jnp.int32)

            def radix_pass(src_dig, src_pay, dst):
                for h in range(256 // _V):
                    hist_v[pl.ds(h * _V, _V)] = jnp.zeros((_V,), jnp.int32)
                for c in range(nv):
                    plsc.addupdate_scatter(hist_v, [src_dig(c)], ones)
                carry_s = jnp.int32(0)
                for h in range(256 // _V):
                    hv = hist_v[pl.ds(h * _V, _V)]
                    inc = plsc.cumsum(hv)
                    off_v[pl.ds(h * _V, _V)] = inc - hv + carry_s
                    carry_s = carry_s + jnp.sum(hv)
                for c in range(nv):
                    d = src_dig(c)
                    cnt, last = plsc.scan_count(d)
                    base = plsc.load_gather(off_v, [d])
                    plsc.store_scatter(dst, [base + cnt - 1], src_pay(c))
                    plsc.addupdate_scatter(off_v, [d], cnt, mask=last)

            def ki_of(c):
                nvv = noise_v[pl.ds(r * L + c * _V, _V)]
                return (nvv * 8388608.0).astype(jnp.int32)

            radix_pass(
                lambda c: ki_of(c) & 255,
                lambda c: ((ki_of(c) >> 8) << 10)
                | (lax.iota(jnp.int32, _V) + c * _V),
                ka)
            radix_pass(
                lambda c: (ka[pl.ds(c * _V, _V)] >> 10) & 255,
                lambda c: ka[pl.ds(c * _V, _V)],
                kb)

            @pl.when(r >= 1)
            def _():
                for j in range(nch):
                    gather_cp(r - 1, j).wait()
                    out_cp(r - 1, j).start()

            radix_pass(
                lambda c: kb[pl.ds(c * _V, _V)] >> 18,
                lambda c: kb[pl.ds(c * _V, _V)],
                ka)

            for q in range(nv):
                sv = ka[pl.ds(q * _V, _V)] & 1023
                kidx = lax.iota(jnp.int32, _V) + q * _V
                plsc.store_scatter(rest4_v, [sv + r * L], kidx)
                plsc.store_scatter(
                    mask4_v, [sv + r * L],
                    jnp.where(kidx >= keep, 1.0, 0.0).astype(jnp.float32))
            for q in range(kv):
                sv = ka[pl.ds(q * _V, _V)] & 1023
                ids_v[pl.ds(r * keep + q * _V, _V)] = sv + (row0 + r) * L

            @pl.when(r >= 1)
            def _():
                for j in range(nch):
                    out_cp(r - 1, j).wait()
            for j in range(nch):
                gather_cp(r, j).start()
            return carry

        lax.fori_loop(0, rows_w, row_body, 0)

        pltpu.sync_copy(mask4_v, mask_hbm.at[pl.ds(row0 * L, rows_w * L)])
        pltpu.sync_copy(rest4_v, rest_hbm.at[pl.ds(row0 * L, rows_w * L)])

        last = rows_w - 1
        for j in range(nch):
            gather_cp(last, j).wait()
            out_cp(last, j).start()
        for j in range(nch):
            out_cp(last, j).wait()

    return k(noise_flat, x2)


def kernel(x):
    N, L, D = x.shape
    keep = int(L * (1 - MASK_FRACTION))
    noise = jax.random.uniform(jax.random.key(42), (N, L), dtype=jnp.float32)
    xm, mask, rest = _sc_mask_and_gather(
        noise.reshape(N * L), x.reshape(N * L, D), N, L, keep)
    return (xm.reshape(N, keep, D),
            mask.reshape(N, L),
            rest.reshape(N, L))
